# baseline (speedup 1.0000x reference)
import jax
import jax.numpy as jnp
from jax import lax
from jax.experimental import pallas as pl
from jax.experimental.pallas import tpu as pltpu
from jax.experimental.pallas import tpu_sc as plsc

N_NODES = 10000
N_EDGES = 320000
D_FEAT = 128

NC = 2
NS = 16
NW = NC * NS
EPW = N_EDGES // NW
CHUNK = 400
NCHUNK = EPW // CHUNK
GROUPS = CHUNK // 16


def _sc_scores(h_hbm, src_hbm, dst_hbm, scores_hbm,
               idx_s, idx_d, srows, drows, scores_v, sem_s, sem_d):
    wid = lax.axis_index("s") * NC + lax.axis_index("c")
    base = wid * EPW
    lanes = lax.iota(jnp.int32, 16)

    def chunk_body(c, carry):
        off = base + c * CHUNK
        pltpu.sync_copy(src_hbm.at[pl.ds(off, CHUNK)], idx_s)
        pltpu.sync_copy(dst_hbm.at[pl.ds(off, CHUNK)], idx_d)
        cp_s = pltpu.async_copy(h_hbm.at[idx_s], srows, sem_s)
        cp_d = pltpu.async_copy(h_hbm.at[idx_d], drows, sem_d)
        cp_s.wait()
        cp_d.wait()

        def g_body(g, carry2):
            rows = lanes + g * 16
            acc = jnp.zeros((16,), jnp.float32)
            for d in range(D_FEAT):
                cols = jnp.full((16,), d, jnp.int32)
                sv = plsc.load_gather(srows, [rows, cols])
                dv = plsc.load_gather(drows, [rows, cols])
                acc = acc + sv * dv
            scores_v[pl.ds(c * CHUNK + g * 16, 16)] = acc
            return carry2

        lax.fori_loop(0, GROUPS, g_body, 0)
        return carry

    lax.fori_loop(0, NCHUNK, chunk_body, 0)
    pltpu.sync_copy(scores_v, scores_hbm.at[pl.ds(base, EPW)])


_sc_call = pl.kernel(
    _sc_scores,
    out_type=jax.ShapeDtypeStruct((N_EDGES,), jnp.float32),
    mesh=plsc.VectorSubcoreMesh(core_axis_name="c", subcore_axis_name="s"),
    compiler_params=pltpu.CompilerParams(needs_layout_passes=False),
    scratch_types=[
        pltpu.VMEM((CHUNK,), jnp.int32),
        pltpu.VMEM((CHUNK,), jnp.int32),
        pltpu.VMEM((CHUNK, D_FEAT), jnp.float32),
        pltpu.VMEM((CHUNK, D_FEAT), jnp.float32),
        pltpu.VMEM((EPW,), jnp.float32),
        pltpu.SemaphoreType.DMA,
        pltpu.SemaphoreType.DMA,
    ],
)


def _norm_body(s_ref, o_ref):
    s = s_ref[...]
    mn = jnp.min(s)
    mx = jnp.max(s)
    o_ref[...] = jnp.where((s - mn) / (mx - mn) == 0.0, 0.0, 1.0)


def kernel(h, edge_index):
    ei = edge_index.astype(jnp.int32)
    scores = _sc_call(h, ei[0], ei[1])
    s2d = scores.reshape(N_EDGES // 128, 128)
    out = pl.pallas_call(
        _norm_body,
        out_shape=jax.ShapeDtypeStruct(s2d.shape, jnp.float32),
    )(s2d)
    return out.reshape(N_EDGES, 1)


# speedup vs baseline: 2.5830x; 2.5830x over previous
import jax
import jax.numpy as jnp
from jax import lax
from jax.experimental import pallas as pl
from jax.experimental.pallas import tpu as pltpu
from jax.experimental.pallas import tpu_sc as plsc

N_NODES = 10000
N_EDGES = 320000
D_FEAT = 128

NC = 2
NS = 16
NW = NC * NS
EPW = N_EDGES // NW
CHUNK = 400
NCHUNK = EPW // CHUNK
GROUPS = CHUNK // 16


def _sc_scores(h_hbm, src_hbm, dst_hbm, scores_hbm,
               idx_s, idx_d, srows, drows, scores_v, sem_s, sem_d):
    wid = lax.axis_index("s") * NC + lax.axis_index("c")
    base = wid * EPW
    lanes = lax.iota(jnp.int32, 16)

    def chunk_body(c, carry):
        off = base + c * CHUNK
        pltpu.sync_copy(src_hbm.at[pl.ds(off, CHUNK)], idx_s)
        pltpu.sync_copy(dst_hbm.at[pl.ds(off, CHUNK)], idx_d)
        cp_s = pltpu.async_copy(h_hbm.at[idx_s], srows, sem_s)
        cp_d = pltpu.async_copy(h_hbm.at[idx_d], drows, sem_d)
        cp_s.wait()
        cp_d.wait()

        def g_body(g, carry2):
            rows = lanes + g * 16
            accs = [jnp.zeros((16,), jnp.float32) for _ in range(4)]
            for d in range(D_FEAT):
                cols = (lanes + d) & (D_FEAT - 1)
                sv = plsc.load_gather(srows, [rows, cols])
                dv = plsc.load_gather(drows, [rows, cols])
                accs[d % 4] = accs[d % 4] + sv * dv
            acc = (accs[0] + accs[1]) + (accs[2] + accs[3])
            scores_v[pl.ds(c * CHUNK + g * 16, 16)] = acc
            return carry2

        lax.fori_loop(0, GROUPS, g_body, 0)
        return carry

    lax.fori_loop(0, NCHUNK, chunk_body, 0)
    pltpu.sync_copy(scores_v, scores_hbm.at[pl.ds(base, EPW)])


_sc_call = pl.kernel(
    _sc_scores,
    out_type=jax.ShapeDtypeStruct((N_EDGES,), jnp.float32),
    mesh=plsc.VectorSubcoreMesh(core_axis_name="c", subcore_axis_name="s"),
    compiler_params=pltpu.CompilerParams(needs_layout_passes=False),
    scratch_types=[
        pltpu.VMEM((CHUNK,), jnp.int32),
        pltpu.VMEM((CHUNK,), jnp.int32),
        pltpu.VMEM((CHUNK, D_FEAT), jnp.float32),
        pltpu.VMEM((CHUNK, D_FEAT), jnp.float32),
        pltpu.VMEM((EPW,), jnp.float32),
        pltpu.SemaphoreType.DMA,
        pltpu.SemaphoreType.DMA,
    ],
)


def _norm_body(s_ref, o_ref):
    s = s_ref[...]
    mn = jnp.min(s)
    mx = jnp.max(s)
    o_ref[...] = jnp.where((s - mn) / (mx - mn) == 0.0, 0.0, 1.0)


def kernel(h, edge_index):
    ei = edge_index.astype(jnp.int32)
    scores = _sc_call(h, ei[0], ei[1])
    s2d = scores.reshape(N_EDGES // 128, 128)
    out = pl.pallas_call(
        _norm_body,
        out_shape=jax.ShapeDtypeStruct(s2d.shape, jnp.float32),
    )(s2d)
    return out.reshape(N_EDGES, 1)
